# baseline (speedup 1.0000x reference)
import functools

import jax
import jax.numpy as jnp
from jax import lax
from jax.experimental import pallas as pl
from jax.experimental.pallas import tpu as pltpu
from jax.experimental.pallas import tpu_sc as plsc

NY, NX, C = 496, 432, 64
M = NY * NX
N_VOX = 120000
NW = 32
BLK = 128
NBLK = M // BLK
NBPW_MAX = 54
RPW_MAX = NBPW_MAX * BLK
NZ = 1024
IDX_PAD = 120320
ROWS_PER_CHUNK = 47
NCHUNK = IDX_PAD // (ROWS_PER_CHUNK * 128)
OOR = 1 << 21
PAIR_MAX = 108 * BLK


def _sc_body(idx_hbm, feat_hbm, out_hbm, owner, partner, shared,
             idxb0, idxb1, rows0, rows1,
             trans, si0, si1, sg0, sg1, sw0, sw1):
    idxb = (idxb0, idxb1)
    rows = (rows0, rows1)
    sems_g = (sg0, sg1)
    sems_w = (sw0, sw1)
    sid = lax.axis_index("s")
    wid = lax.axis_index("c") * 16 + sid
    lane = lax.iota(jnp.int32, 16)
    start_block = 52 * wid + 2 * jnp.minimum(wid, 5)
    nb = jnp.where(wid < 5, 54, 52)
    wpair = wid & ~1
    grp = wid & 1
    nb_a = jnp.where(wpair < 5, 54, 52)
    nb_pair = nb_a + jnp.where(wpair + 1 < 5, 54, 52)
    lo = (52 * wpair + 2 * jnp.minimum(wpair, 5)) * BLK
    myoff = jnp.where(grp == 0, 0, nb_a * BLK)
    rpw_u = (nb_pair * BLK).astype(jnp.uint32)

    @plsc.parallel_loop(0, PAIR_MAX // 16, unroll=8)
    def initb(t):
        owner[pl.ds(t * 16, 16)] = jnp.full((16,), -1, jnp.int32)

    sems_i = (si0, si1)
    kgrp = grp * ROWS_PER_CHUNK
    hs = [pltpu.async_copy(idx_hbm.at[pl.ds(kgrp, ROWS_PER_CHUNK)], idxb[0],
                           si0), None]
    for k in range(NCHUNK // 2):
        p = k & 1
        if k + 1 < NCHUNK // 2:
            hs[1 - p] = pltpu.async_copy(
                idx_hbm.at[pl.ds(kgrp + (k + 1) * 2 * ROWS_PER_CHUNK,
                                 ROWS_PER_CHUNK)],
                idxb[1 - p], sems_i[1 - p])
        hs[p].wait()

        def scan_t(t, carry, p=p, kbase=k * 2 * ROWS_PER_CHUNK * 128):
            rowb = kbase + kgrp * 128 + t * 128
            for j in range(8):
                idx16 = idxb[p][t, pl.ds(j * 16, 16)]
                loc = idx16 - lo
                inr = plsc.bitcast(loc, jnp.uint32) < rpw_u
                ivec = rowb + j * 16 + lane
                _, lastm = plsc.scan_count(loc, mask=inr)
                plsc.store_scatter(owner, [loc], ivec, mask=lastm & inr)
            return carry
        lax.fori_loop(0, ROWS_PER_CHUNK, scan_t, 0)

    pltpu.sync_copy(owner, shared.at[sid])
    plsc.subcore_barrier()
    pltpu.sync_copy(
        shared.at[sid + 1 - 2 * grp, pl.ds(myoff, RPW_MAX)], partner)

    @plsc.parallel_loop(0, RPW_MAX // 16, unroll=8)
    def mergeb(t):
        a = owner[pl.ds(myoff + t * 16, 16)]
        b = partner[pl.ds(t * 16, 16)]
        m = jnp.maximum(a, b)
        sent = N_VOX + ((t * 16 + lane) & (NZ - 1))
        owner[pl.ds(myoff + t * 16, 16)] = jnp.where(m < 0, sent, m)

    cols = [cg * 16 + lane for cg in range(4)]
    zero16 = jnp.zeros((16,), jnp.int32)

    def gather_blk(b, p):
        pltpu.async_copy(
            feat_hbm.at[owner.at[pl.ds(myoff + b * BLK, BLK)]],
            rows[p], sems_g[p])

    def wait_gather(p):
        pltpu.make_async_copy(
            feat_hbm.at[owner.at[pl.ds(0, BLK)]], rows[p], sems_g[p]).wait()

    gather_blk(0, 0)
    gather_blk(1, 1)

    def pair(j, carry):
        for p in range(2):
            b = 2 * j + p
            wait_gather(p)

            @pl.when(j > 0)
            def _(p=p):
                pltpu.make_async_copy(
                    trans.at[:, pl.ds(p * BLK, BLK)],
                    out_hbm.at[:, pl.ds(0, BLK)], sems_w[p]).wait()

            @plsc.parallel_loop(0, BLK, unroll=8)
            def transp(r, p=p):
                rs = zero16 + (p * BLK + r)
                for cg in range(4):
                    v = rows[p][r, pl.ds(cg * 16, 16)]
                    plsc.store_scatter(trans, [cols[cg], rs], v)

            pltpu.async_copy(
                trans.at[:, pl.ds(p * BLK, BLK)],
                out_hbm.at[:, pl.ds((start_block + b) * BLK, BLK)], sems_w[p])

            @pl.when(b + 2 < nb)
            def _(b=b, p=p):
                gather_blk(b + 2, p)
        return carry
    lax.fori_loop(0, nb // 2, pair, 0)

    for p in range(2):
        pltpu.make_async_copy(
            trans.at[:, pl.ds(p * BLK, BLK)],
            out_hbm.at[:, pl.ds(0, BLK)], sems_w[p]).wait()


_sc_scatter = functools.partial(
    pl.kernel,
    out_type=jax.ShapeDtypeStruct((C, M), jnp.float32),
    mesh=plsc.VectorSubcoreMesh(core_axis_name="c", subcore_axis_name="s"),
    compiler_params=pltpu.CompilerParams(
        needs_layout_passes=False, use_tc_tiling_on_sc=False),
    scratch_types=[
        pltpu.VMEM((PAIR_MAX,), jnp.int32),
        pltpu.VMEM((RPW_MAX,), jnp.int32),
        pltpu.VMEM_SHARED((16, PAIR_MAX), jnp.int32),
        pltpu.VMEM((ROWS_PER_CHUNK, 128), jnp.int32),
        pltpu.VMEM((ROWS_PER_CHUNK, 128), jnp.int32),
        pltpu.VMEM((BLK, C), jnp.float32),
        pltpu.VMEM((BLK, C), jnp.float32),
        pltpu.VMEM((C, 2 * BLK), jnp.float32),
        pltpu.SemaphoreType.DMA,
        pltpu.SemaphoreType.DMA,
        pltpu.SemaphoreType.DMA,
        pltpu.SemaphoreType.DMA,
        pltpu.SemaphoreType.DMA,
        pltpu.SemaphoreType.DMA,
    ],
)(_sc_body)


def kernel(voxel_features, coors):
    idx = coors[:, 1] * NX + coors[:, 2]
    idx_pad = jnp.concatenate(
        [idx, jnp.full((IDX_PAD - N_VOX,), OOR, jnp.int32)]).reshape(-1, 128)
    feat_ext = jnp.concatenate(
        [voxel_features, jnp.zeros((NZ, C), jnp.float32)], axis=0)
    canvas = _sc_scatter(idx_pad, feat_ext)
    return (jnp.reshape(canvas, (1, C, NY, NX)),)


# speedup vs baseline: 1.2462x; 1.2462x over previous
import functools

import jax
import jax.numpy as jnp
from jax import lax
from jax.experimental import pallas as pl
from jax.experimental.pallas import tpu as pltpu
from jax.experimental.pallas import tpu_sc as plsc

NY, NX, C = 496, 432, 64
M = NY * NX
N_VOX = 120000
NW = 32
BLK = 128
NBLK = M // BLK
NBPW_MAX = 54
RPW_MAX = NBPW_MAX * BLK
NZ = 1024
IDX_PAD = 120320
ROWS_PER_CHUNK = 47
NCHUNK = IDX_PAD // (ROWS_PER_CHUNK * 128)
OOR = 1 << 21
PAIR_MAX = 108 * BLK


def _sc_body(idx_hbm, feat_hbm, out_hbm, owner, partner, shared,
             idxb0, idxb1, rows0, rows1,
             trans, si0, si1, sg0, sg1, sw0, sw1):
    idxb = (idxb0, idxb1)
    rows = (rows0, rows1)
    sems_g = (sg0, sg1)
    sems_w = (sw0, sw1)
    sid = lax.axis_index("s")
    wid = lax.axis_index("c") * 16 + sid
    lane = lax.iota(jnp.int32, 16)
    start_block = 52 * wid + 2 * jnp.minimum(wid, 5)
    nb = jnp.where(wid < 5, 54, 52)
    wpair = wid & ~1
    grp = wid & 1
    nb_a = jnp.where(wpair < 5, 54, 52)
    nb_pair = nb_a + jnp.where(wpair + 1 < 5, 54, 52)
    lo = (52 * wpair + 2 * jnp.minimum(wpair, 5)) * BLK
    myoff = jnp.where(grp == 0, 0, nb_a * BLK)
    rpw_u = (nb_pair * BLK).astype(jnp.uint32)

    @plsc.parallel_loop(0, PAIR_MAX // 16, unroll=8)
    def initb(t):
        owner[pl.ds(t * 16, 16)] = jnp.full((16,), -1, jnp.int32)

    sems_i = (si0, si1)
    kgrp = grp * ROWS_PER_CHUNK
    hs = [pltpu.async_copy(idx_hbm.at[pl.ds(kgrp, ROWS_PER_CHUNK)], idxb[0],
                           si0), None]
    for k in range(NCHUNK // 2):
        p = k & 1
        if k + 1 < NCHUNK // 2:
            hs[1 - p] = pltpu.async_copy(
                idx_hbm.at[pl.ds(kgrp + (k + 1) * 2 * ROWS_PER_CHUNK,
                                 ROWS_PER_CHUNK)],
                idxb[1 - p], sems_i[1 - p])
        hs[p].wait()

        def scan_t(t, carry, p=p, kbase=k * 2 * ROWS_PER_CHUNK * 128):
            rowb = kbase + kgrp * 128 + t * 128
            for j in range(8):
                idx16 = idxb[p][t, pl.ds(j * 16, 16)]
                loc = idx16 - lo
                inr = plsc.bitcast(loc, jnp.uint32) < rpw_u
                ivec = rowb + j * 16 + lane
                _, lastm = plsc.scan_count(loc, mask=inr)
                plsc.store_scatter(owner, [loc], ivec, mask=lastm & inr)
            return carry
        lax.fori_loop(0, ROWS_PER_CHUNK, scan_t, 0)

    pltpu.sync_copy(owner, shared.at[sid])
    plsc.subcore_barrier()
    pltpu.sync_copy(
        shared.at[sid + 1 - 2 * grp, pl.ds(myoff, RPW_MAX)], partner)

    @plsc.parallel_loop(0, RPW_MAX // 16, unroll=8)
    def mergeb(t):
        a = owner[pl.ds(myoff + t * 16, 16)]
        b = partner[pl.ds(t * 16, 16)]
        m = jnp.maximum(a, b)
        sent = N_VOX + ((t * 16 + lane) & (NZ - 1))
        owner[pl.ds(myoff + t * 16, 16)] = jnp.where(m < 0, sent, m)

    cols = [cg * 16 + lane for cg in range(4)]
    zero16 = jnp.zeros((16,), jnp.int32)

    def gather_blk(b, p):
        pltpu.async_copy(
            feat_hbm.at[owner.at[pl.ds(myoff + b * BLK, BLK)]],
            rows[p], sems_g[p])

    def wait_gather(p):
        pltpu.make_async_copy(
            feat_hbm.at[owner.at[pl.ds(0, BLK)]], rows[p], sems_g[p]).wait()

    gather_blk(0, 0)
    gather_blk(1, 1)

    def pair(j, carry):
        for p in range(2):
            b = 2 * j + p
            wait_gather(p)

            @pl.when(j > 0)
            def _(p=p):
                pltpu.make_async_copy(
                    trans.at[:, pl.ds(p * BLK, BLK)],
                    out_hbm.at[:, pl.ds(0, BLK)], sems_w[p]).wait()

            @plsc.parallel_loop(0, BLK, unroll=8)
            def transp(r, p=p):
                rs = zero16 + (p * BLK + r)
                for cg in range(4):
                    v = rows[p][r, pl.ds(cg * 16, 16)]
                    plsc.store_scatter(trans, [cols[cg], rs], v)

            pltpu.async_copy(
                trans.at[:, pl.ds(p * BLK, BLK)],
                out_hbm.at[:, pl.ds((start_block + b) * BLK, BLK)], sems_w[p])

            @pl.when(b + 2 < nb)
            def _(b=b, p=p):
                gather_blk(b + 2, p)
        return carry
    lax.fori_loop(0, nb // 2, pair, 0)

    for p in range(2):
        pltpu.make_async_copy(
            trans.at[:, pl.ds(p * BLK, BLK)],
            out_hbm.at[:, pl.ds(0, BLK)], sems_w[p]).wait()


_sc_scatter = functools.partial(
    pl.kernel,
    out_type=jax.ShapeDtypeStruct((C, M), jnp.float32),
    mesh=plsc.VectorSubcoreMesh(core_axis_name="c", subcore_axis_name="s"),
    compiler_params=pltpu.CompilerParams(
        needs_layout_passes=False, use_tc_tiling_on_sc=False),
    scratch_types=[
        pltpu.VMEM((PAIR_MAX,), jnp.int32),
        pltpu.VMEM((RPW_MAX,), jnp.int32),
        pltpu.VMEM_SHARED((16, PAIR_MAX), jnp.int32),
        pltpu.VMEM((ROWS_PER_CHUNK, 128), jnp.int32),
        pltpu.VMEM((ROWS_PER_CHUNK, 128), jnp.int32),
        pltpu.VMEM((BLK, C), jnp.float32),
        pltpu.VMEM((BLK, C), jnp.float32),
        pltpu.VMEM((C, 2 * BLK + 1), jnp.float32),
        pltpu.SemaphoreType.DMA,
        pltpu.SemaphoreType.DMA,
        pltpu.SemaphoreType.DMA,
        pltpu.SemaphoreType.DMA,
        pltpu.SemaphoreType.DMA,
        pltpu.SemaphoreType.DMA,
    ],
)(_sc_body)


def kernel(voxel_features, coors):
    idx = coors[:, 1] * NX + coors[:, 2]
    idx_pad = jnp.concatenate(
        [idx, jnp.full((IDX_PAD - N_VOX,), OOR, jnp.int32)]).reshape(-1, 128)
    feat_ext = jnp.concatenate(
        [voxel_features, jnp.zeros((NZ, C), jnp.float32)], axis=0)
    canvas = _sc_scatter(idx_pad, feat_ext)
    return (jnp.reshape(canvas, (1, C, NY, NX)),)


# speedup vs baseline: 1.3120x; 1.0528x over previous
import functools

import jax
import jax.numpy as jnp
from jax import lax
from jax.experimental import pallas as pl
from jax.experimental.pallas import tpu as pltpu
from jax.experimental.pallas import tpu_sc as plsc

NY, NX, C = 496, 432, 64
M = NY * NX
N_VOX = 120000
NW = 32
BLK = 128
NBLK = M // BLK
NBPW_MAX = 54
RPW_MAX = NBPW_MAX * BLK
NZ = 1024
IDX_PAD = 120320
ROWS_PER_CHUNK = 47
NCHUNK = IDX_PAD // (ROWS_PER_CHUNK * 128)
OOR = 1 << 21
PAIR_MAX = 212 * BLK


def _sc_body(idx_hbm, feat_hbm, out_hbm, owner, partner, shared,
             idxb0, idxb1, rows0, rows1,
             trans, si0, si1, sg0, sg1, sw0, sw1):
    idxb = (idxb0, idxb1)
    rows = (rows0, rows1)
    sems_g = (sg0, sg1)
    sems_w = (sw0, sw1)
    sid = lax.axis_index("s")
    wid = lax.axis_index("c") * 16 + sid
    lane = lax.iota(jnp.int32, 16)
    start_block = 52 * wid + 2 * jnp.minimum(wid, 5)
    nb = jnp.where(wid < 5, 54, 52)
    wbase = wid & ~3
    grp = wid & 3

    def _nb(w):
        return jnp.where(w < 5, 54, 52)

    def _sb(w):
        return 52 * w + 2 * jnp.minimum(w, 5)

    nb_quad = _nb(wbase) + _nb(wbase + 1) + _nb(wbase + 2) + _nb(wbase + 3)
    lo = _sb(wbase) * BLK
    myoff = (start_block - _sb(wbase)) * BLK
    rpw_u = (nb_quad * BLK).astype(jnp.uint32)

    @plsc.parallel_loop(0, PAIR_MAX // 16, unroll=8)
    def initb(t):
        owner[pl.ds(t * 16, 16)] = jnp.full((16,), -1, jnp.int32)

    sems_i = (si0, si1)
    kgrp = grp * ROWS_PER_CHUNK
    hs = [pltpu.async_copy(idx_hbm.at[pl.ds(kgrp, ROWS_PER_CHUNK)], idxb[0],
                           si0), None]
    for k in range(NCHUNK // 4):
        p = k & 1
        if k + 1 < NCHUNK // 4:
            hs[1 - p] = pltpu.async_copy(
                idx_hbm.at[pl.ds(kgrp + (k + 1) * 4 * ROWS_PER_CHUNK,
                                 ROWS_PER_CHUNK)],
                idxb[1 - p], sems_i[1 - p])
        hs[p].wait()

        def scan_t(t, carry, p=p, kbase=k * 4 * ROWS_PER_CHUNK * 128):
            rowb = kbase + kgrp * 128 + t * 128
            for j in range(8):
                idx16 = idxb[p][t, pl.ds(j * 16, 16)]
                loc = idx16 - lo
                inr = plsc.bitcast(loc, jnp.uint32) < rpw_u
                ivec = rowb + j * 16 + lane
                _, lastm = plsc.scan_count(loc, mask=inr)
                plsc.store_scatter(owner, [loc], ivec, mask=lastm & inr)
            return carry
        lax.fori_loop(0, ROWS_PER_CHUNK, scan_t, 0)

    pltpu.sync_copy(owner, shared.at[sid])
    plsc.subcore_barrier()
    sbase = sid & ~3
    for q in range(3):
        psid = sbase + jnp.where(grp <= q, q + 1, q)
        pltpu.sync_copy(shared.at[psid, pl.ds(myoff, RPW_MAX)], partner)

        if q < 2:
            @plsc.parallel_loop(0, RPW_MAX // 16, unroll=8)
            def mergeb(t):
                a = owner[pl.ds(myoff + t * 16, 16)]
                b = partner[pl.ds(t * 16, 16)]
                owner[pl.ds(myoff + t * 16, 16)] = jnp.maximum(a, b)
        else:
            @plsc.parallel_loop(0, RPW_MAX // 16, unroll=8)
            def mergeb(t):
                a = owner[pl.ds(myoff + t * 16, 16)]
                b = partner[pl.ds(t * 16, 16)]
                m = jnp.maximum(a, b)
                sent = N_VOX + ((t * 16 + lane) & (NZ - 1))
                owner[pl.ds(myoff + t * 16, 16)] = jnp.where(m < 0, sent, m)

    cols = [cg * 16 + lane for cg in range(4)]
    zero16 = jnp.zeros((16,), jnp.int32)

    def gather_blk(b, p):
        pltpu.async_copy(
            feat_hbm.at[owner.at[pl.ds(myoff + b * BLK, BLK)]],
            rows[p], sems_g[p])

    def wait_gather(p):
        pltpu.make_async_copy(
            feat_hbm.at[owner.at[pl.ds(0, BLK)]], rows[p], sems_g[p]).wait()

    gather_blk(0, 0)
    gather_blk(1, 1)

    def pair(j, carry):
        for p in range(2):
            b = 2 * j + p
            wait_gather(p)

            @pl.when(j > 0)
            def _(p=p):
                pltpu.make_async_copy(
                    trans.at[:, pl.ds(p * BLK, BLK)],
                    out_hbm.at[:, pl.ds(0, BLK)], sems_w[p]).wait()

            @plsc.parallel_loop(0, BLK, unroll=8)
            def transp(r, p=p):
                rs = zero16 + (p * BLK + r)
                for cg in range(4):
                    v = rows[p][r, pl.ds(cg * 16, 16)]
                    plsc.store_scatter(trans, [cols[cg], rs], v)

            pltpu.async_copy(
                trans.at[:, pl.ds(p * BLK, BLK)],
                out_hbm.at[:, pl.ds((start_block + b) * BLK, BLK)], sems_w[p])

            @pl.when(b + 2 < nb)
            def _(b=b, p=p):
                gather_blk(b + 2, p)
        return carry
    lax.fori_loop(0, nb // 2, pair, 0)

    for p in range(2):
        pltpu.make_async_copy(
            trans.at[:, pl.ds(p * BLK, BLK)],
            out_hbm.at[:, pl.ds(0, BLK)], sems_w[p]).wait()


_sc_scatter = functools.partial(
    pl.kernel,
    out_type=jax.ShapeDtypeStruct((C, M), jnp.float32),
    mesh=plsc.VectorSubcoreMesh(core_axis_name="c", subcore_axis_name="s"),
    compiler_params=pltpu.CompilerParams(
        needs_layout_passes=False, use_tc_tiling_on_sc=False),
    scratch_types=[
        pltpu.VMEM((PAIR_MAX,), jnp.int32),
        pltpu.VMEM((RPW_MAX,), jnp.int32),
        pltpu.VMEM_SHARED((16, PAIR_MAX), jnp.int32),
        pltpu.VMEM((ROWS_PER_CHUNK, 128), jnp.int32),
        pltpu.VMEM((ROWS_PER_CHUNK, 128), jnp.int32),
        pltpu.VMEM((BLK, C), jnp.float32),
        pltpu.VMEM((BLK, C), jnp.float32),
        pltpu.VMEM((C, 2 * BLK + 1), jnp.float32),
        pltpu.SemaphoreType.DMA,
        pltpu.SemaphoreType.DMA,
        pltpu.SemaphoreType.DMA,
        pltpu.SemaphoreType.DMA,
        pltpu.SemaphoreType.DMA,
        pltpu.SemaphoreType.DMA,
    ],
)(_sc_body)


def kernel(voxel_features, coors):
    idx = coors[:, 1] * NX + coors[:, 2]
    idx_pad = jnp.concatenate(
        [idx, jnp.full((IDX_PAD - N_VOX,), OOR, jnp.int32)]).reshape(-1, 128)
    feat_ext = jnp.concatenate(
        [voxel_features, jnp.zeros((NZ, C), jnp.float32)], axis=0)
    canvas = _sc_scatter(idx_pad, feat_ext)
    return (jnp.reshape(canvas, (1, C, NY, NX)),)
